# Initial kernel scaffold; baseline (speedup 1.0000x reference)
import jax
import jax.numpy as jnp
from jax.experimental import pallas as pl


def kernel(node_feats, edge_index, W_msg, b_msg, W_out, b_out):
    raise NotImplementedError("write your pallas kernel here")


# speedup vs baseline: 7.6920x; 7.6920x over previous
import functools

import jax
import jax.numpy as jnp
from jax import lax
from jax.experimental import pallas as pl
from jax.experimental.pallas import tpu as pltpu
from jax.experimental.pallas import tpu_sc as plsc

N_NODES = 10000
N_EDGES = 320000
D = 128

NC = 2
NS = 16
NW = NC * NS
EDGES_PER_TILE = N_EDGES // NW
CHUNK = 200
NCHUNK = EDGES_PER_TILE // CHUNK
N_PAD = 10240
ROWS_PER_TILE = N_PAD // NS

_BLK = 2000


def _mm_relu_body(x_ref, w_ref, b_ref, o_ref):
    o_ref[...] = jnp.maximum(
        jnp.dot(x_ref[...], w_ref[...], preferred_element_type=jnp.float32)
        + b_ref[...], 0.0)


def _mm_relu(x, w, b):
    n = x.shape[0]
    return pl.pallas_call(
        _mm_relu_body,
        grid=(n // _BLK,),
        in_specs=[
            pl.BlockSpec((_BLK, D), lambda i: (i, 0)),
            pl.BlockSpec((D, D), lambda i: (0, 0)),
            pl.BlockSpec((1, D), lambda i: (0, 0)),
        ],
        out_specs=pl.BlockSpec((_BLK, D), lambda i: (i, 0)),
        out_shape=jax.ShapeDtypeStruct((n, D), jnp.float32),
    )(x, w, b.reshape(1, D))


def _final_mm_body(p0_ref, p1_ref, w_ref, b_ref, o_ref):
    agg = p0_ref[...] + p1_ref[...]
    o_ref[...] = (
        jnp.dot(agg, w_ref[...], preferred_element_type=jnp.float32)
        + b_ref[...])


def _final_mm(p0, p1, w, b):
    n = N_NODES
    return pl.pallas_call(
        _final_mm_body,
        grid=(n // _BLK,),
        in_specs=[
            pl.BlockSpec((_BLK, D), lambda i: (i, 0)),
            pl.BlockSpec((_BLK, D), lambda i: (i, 0)),
            pl.BlockSpec((D, D), lambda i: (0, 0)),
            pl.BlockSpec((1, D), lambda i: (0, 0)),
        ],
        out_specs=pl.BlockSpec((_BLK, D), lambda i: (i, 0)),
        out_shape=jax.ShapeDtypeStruct((n, D), jnp.float32),
    )(p0, p1, w, b.reshape(1, D))


def _sc_edge_body(src_hbm, dst_hbm, z_hbm, zeros_hbm, out_hbm,
                  src_v, dst_v, rows_v, agg_sh, sem):
    cid = lax.axis_index("c")
    sid = lax.axis_index("s")

    row_base = sid * ROWS_PER_TILE
    pltpu.sync_copy(zeros_hbm, agg_sh.at[pl.ds(row_base, ROWS_PER_TILE)])
    plsc.subcore_barrier()

    wid = cid * NS + sid
    ebase = wid * EDGES_PER_TILE

    @pl.loop(0, NCHUNK)
    def _chunk(i):
        off = ebase + i * CHUNK
        pltpu.sync_copy(src_hbm.at[pl.ds(off, CHUNK)], src_v)
        pltpu.sync_copy(dst_hbm.at[pl.ds(off, CHUNK)], dst_v)
        pltpu.async_copy(z_hbm.at[src_v], rows_v, sem).wait()
        pltpu.sync_copy(rows_v, agg_sh.at[dst_v], add=True)

    plsc.subcore_barrier()
    pltpu.sync_copy(agg_sh.at[pl.ds(row_base, ROWS_PER_TILE)],
                    out_hbm.at[cid, pl.ds(row_base, ROWS_PER_TILE)])


_sc_edge = functools.partial(
    pl.kernel,
    out_type=jax.ShapeDtypeStruct((NC, N_PAD, D), jnp.float32),
    mesh=plsc.VectorSubcoreMesh(
        core_axis_name="c", subcore_axis_name="s",
        num_cores=NC, num_subcores=NS),
    scratch_types=[
        pltpu.VMEM((CHUNK,), jnp.int32),
        pltpu.VMEM((CHUNK,), jnp.int32),
        pltpu.VMEM((CHUNK, D), jnp.float32),
        pltpu.VMEM_SHARED((N_PAD, D), jnp.float32),
        pltpu.SemaphoreType.DMA,
    ],
)(_sc_edge_body)


def kernel(node_feats, edge_index, W_msg, b_msg, W_out, b_out):
    src = edge_index[0].astype(jnp.int32)
    dst = edge_index[1].astype(jnp.int32)
    z = _mm_relu(node_feats, W_msg, b_msg)
    zeros = jnp.zeros((ROWS_PER_TILE, D), jnp.float32)
    partials = _sc_edge(src, dst, z, zeros)
    return _final_mm(partials[0], partials[1], W_out, b_out)
